# Initial kernel scaffold; baseline (speedup 1.0000x reference)
import jax
import jax.numpy as jnp
from jax.experimental import pallas as pl


def kernel(x, edge_index, edge_type, comp0, bases0, root0, bias0, bn_gamma, bn_beta, bn_mean, bn_var, comp1, bases1, root1, bias1):
    raise NotImplementedError("write your pallas kernel here")


# speedup vs baseline: 5.9370x; 5.9370x over previous
import functools

import jax
import jax.numpy as jnp
from jax import lax
from jax.experimental import pallas as pl
from jax.experimental.pallas import tpu as pltpu
from jax.experimental.pallas import tpu_sc as plsc

N = 10000
E = 320000
F = 128
R = 5
NB = 4
EPS_BN = 1e-5
EPS_NORM = 1e-12

NC = 2
NS = 16
NW = NC * NS
EPW = E // NW
CH = 80
NCHUNK = EPW // CH

KROWS = 3200
KPAD = KROWS * 16
ROW_A = 624
ROW_TAIL = N - NS * ROW_A
KR_PER_TILE = KROWS // NS

_mesh = plsc.VectorSubcoreMesh(core_axis_name="c", subcore_axis_name="s",
                               num_cores=NC, num_subcores=NS)
_SC_PARAMS = pltpu.CompilerParams(needs_layout_passes=False)


def _cnt_body(dst_hbm, et_hbm, zero_hbm, out_hbm,
              dst_v, et_v, key_v, ones_v, cnt_sh):
  c = lax.axis_index("c")
  s = lax.axis_index("s")
  wid = s * NC + c

  kst = pl.multiple_of(s * (KPAD // NS), 8)
  pltpu.sync_copy(zero_hbm.at[pl.ds(kst, KPAD // NS)],
                  cnt_sh.at[pl.ds(kst, KPAD // NS)])
  for g in range(CH // 16):
    ones_v[pl.ds(g * 16, 16)] = jnp.full((16,), 1.0, jnp.float32)
  plsc.subcore_barrier()

  def chunk(t, carry):
    base = pl.multiple_of(wid * EPW + t * CH, 8)
    pltpu.sync_copy(dst_hbm.at[pl.ds(base, CH)], dst_v)
    pltpu.sync_copy(et_hbm.at[pl.ds(base, CH)], et_v)
    for g in range(CH // 16):
      d16 = dst_v[pl.ds(g * 16, 16)]
      e16 = et_v[pl.ds(g * 16, 16)]
      key_v[pl.ds(g * 16, 16)] = d16 * R + e16
    pltpu.sync_copy(ones_v, cnt_sh.at[key_v], add=True)
    return carry

  lax.fori_loop(0, NCHUNK, chunk, 0)
  plsc.subcore_barrier()

  pltpu.sync_copy(cnt_sh.at[pl.ds(kst, KPAD // NS)],
                  out_hbm.at[c, pl.ds(kst, KPAD // NS)])


_cnt_call = pl.kernel(
    _cnt_body,
    out_type=jax.ShapeDtypeStruct((NC, KPAD), jnp.float32),
    mesh=_mesh,
    compiler_params=_SC_PARAMS,
    scratch_types=[
        pltpu.VMEM((CH,), jnp.int32),
        pltpu.VMEM((CH,), jnp.int32),
        pltpu.VMEM((CH,), jnp.int32),
        pltpu.VMEM((CH,), jnp.float32),
        pltpu.VMEM_SHARED((KPAD,), jnp.float32),
    ],
)


def _pass_body(p_hbm, inv_hbm, src_hbm, et_hbm, dst_hbm, zero_hbm, out_hbm,
               src_v, et_v, dst_v, ridx_v, key_v, w_v, rows_v,
               inv_sh, acc_sh, sem):
  c = lax.axis_index("c")
  s = lax.axis_index("s")
  wid = s * NC + c

  rst = pl.multiple_of(s * ROW_A, 8)
  pltpu.sync_copy(zero_hbm.at[pl.ds(rst, ROW_A)],
                  acc_sh.at[pl.ds(rst, ROW_A)])
  @pl.when(s == NS - 1)
  def _zero_tail():
    pltpu.sync_copy(zero_hbm.at[pl.ds(NS * ROW_A, ROW_TAIL)],
                    acc_sh.at[pl.ds(NS * ROW_A, ROW_TAIL)])
  @pl.when(s == 0)
  def _stage_inv():
    pltpu.sync_copy(inv_hbm, inv_sh)
  plsc.subcore_barrier()

  def chunk(t, carry):
    base = pl.multiple_of(wid * EPW + t * CH, 8)
    pltpu.sync_copy(src_hbm.at[pl.ds(base, CH)], src_v)
    pltpu.sync_copy(et_hbm.at[pl.ds(base, CH)], et_v)
    pltpu.sync_copy(dst_hbm.at[pl.ds(base, CH)], dst_v)
    iota16 = lax.iota(jnp.int32, 16)
    for g in range(CH // 16):
      s16 = src_v[pl.ds(g * 16, 16)]
      e16 = et_v[pl.ds(g * 16, 16)]
      d16 = dst_v[pl.ds(g * 16, 16)]
      ridx_v[pl.ds(g * 16, 16)] = e16 * N + s16
      key_v[pl.ds(g * 16, 16)] = d16 * R + e16
    rows_dma = pltpu.async_copy(p_hbm.at[ridx_v], rows_v, sem)
    pltpu.sync_copy(inv_sh.at[key_v], w_v)
    rows_dma.wait()
    def scale(i, carry2):
      rsplat = jnp.full((16,), i, jnp.int32)
      w16 = plsc.load_gather(w_v, [rsplat])
      for j in range(F // 16):
        col = iota16 + (j * 16)
        v = plsc.load_gather(rows_v, [rsplat, col])
        plsc.store_scatter(rows_v, [rsplat, col], v * w16)
      return carry2
    lax.fori_loop(0, CH, scale, 0)
    pltpu.sync_copy(rows_v, acc_sh.at[dst_v], add=True)
    return carry

  lax.fori_loop(0, NCHUNK, chunk, 0)
  plsc.subcore_barrier()

  pltpu.sync_copy(acc_sh.at[pl.ds(rst, ROW_A)],
                  out_hbm.at[c, pl.ds(rst, ROW_A)])
  @pl.when(s == NS - 1)
  def _out_tail():
    pltpu.sync_copy(acc_sh.at[pl.ds(NS * ROW_A, ROW_TAIL)],
                    out_hbm.at[c, pl.ds(NS * ROW_A, ROW_TAIL)])


_pass_call = pl.kernel(
    _pass_body,
    out_type=jax.ShapeDtypeStruct((NC, N, F), jnp.float32),
    mesh=_mesh,
    compiler_params=_SC_PARAMS,
    scratch_types=[
        pltpu.VMEM((CH,), jnp.int32),
        pltpu.VMEM((CH,), jnp.int32),
        pltpu.VMEM((CH,), jnp.int32),
        pltpu.VMEM((CH,), jnp.int32),
        pltpu.VMEM((CH,), jnp.int32),
        pltpu.VMEM((CH,), jnp.float32),
        pltpu.VMEM((CH, F), jnp.float32),
        pltpu.VMEM_SHARED((KPAD,), jnp.float32),
        pltpu.VMEM_SHARED((N, F), jnp.float32),
        pltpu.SemaphoreType.DMA,
    ],
)


_BLK = 1000
_GRID = N // _BLK
_DOT = functools.partial(lax.dot, precision=lax.Precision.HIGHEST,
                         preferred_element_type=jnp.float32)


def _tc1_body(x_ref, comp_ref, bases_ref, cnt_ref, p_ref, inv_ref):
  csum = cnt_ref[0] + cnt_ref[1]
  inv_ref[...] = 1.0 / jnp.maximum(csum, 1.0)
  xb = x_ref[...]
  for r in range(R):
    w = (comp_ref[r, 0] * bases_ref[0] + comp_ref[r, 1] * bases_ref[1]
         + comp_ref[r, 2] * bases_ref[2] + comp_ref[r, 3] * bases_ref[3])
    p_ref[r] = _DOT(xb, w)


def _tc1(x, comp0, bases0, cnt2):
  return pl.pallas_call(
      _tc1_body,
      grid=(_GRID,),
      in_specs=[
          pl.BlockSpec((_BLK, F), lambda i: (i, 0)),
          pl.BlockSpec((R, NB), lambda i: (0, 0)),
          pl.BlockSpec((NB, F, F), lambda i: (0, 0, 0)),
          pl.BlockSpec((NC, KPAD // 128, 128), lambda i: (0, 0, 0)),
      ],
      out_specs=[
          pl.BlockSpec((R, _BLK, F), lambda i: (0, i, 0)),
          pl.BlockSpec((KPAD // 128, 128), lambda i: (0, 0)),
      ],
      out_shape=[
          jax.ShapeDtypeStruct((R, N, F), jnp.float32),
          jax.ShapeDtypeStruct((KPAD // 128, 128), jnp.float32),
      ],
  )(x, comp0, bases0, cnt2)


def _tc2_body(x_ref, a_ref, root_ref, bias_ref, sc_ref, sh_ref,
              comp_ref, bases_ref, p_ref, h_ref):
  z = _DOT(x_ref[...], root_ref[...]) + bias_ref[...]
  z = z + a_ref[0] + a_ref[1]
  h = jnp.maximum(z * sc_ref[...] + sh_ref[...], 0.0)
  h_ref[...] = h
  for r in range(R):
    w = (comp_ref[r, 0] * bases_ref[0] + comp_ref[r, 1] * bases_ref[1]
         + comp_ref[r, 2] * bases_ref[2] + comp_ref[r, 3] * bases_ref[3])
    p_ref[r] = _DOT(h, w)


def _tc2(x, a0, root0, bias0, bnscale, bnshift, comp1, bases1):
  return pl.pallas_call(
      _tc2_body,
      grid=(_GRID,),
      in_specs=[
          pl.BlockSpec((_BLK, F), lambda i: (i, 0)),
          pl.BlockSpec((NC, _BLK, F), lambda i: (0, i, 0)),
          pl.BlockSpec((F, F), lambda i: (0, 0)),
          pl.BlockSpec((F,), lambda i: (0,)),
          pl.BlockSpec((F,), lambda i: (0,)),
          pl.BlockSpec((F,), lambda i: (0,)),
          pl.BlockSpec((R, NB), lambda i: (0, 0)),
          pl.BlockSpec((NB, F, F), lambda i: (0, 0, 0)),
      ],
      out_specs=[
          pl.BlockSpec((R, _BLK, F), lambda i: (0, i, 0)),
          pl.BlockSpec((_BLK, F), lambda i: (i, 0)),
      ],
      out_shape=[
          jax.ShapeDtypeStruct((R, N, F), jnp.float32),
          jax.ShapeDtypeStruct((N, F), jnp.float32),
      ],
  )(x, a0, root0, bias0, bnscale, bnshift, comp1, bases1)


def _tc3_body(h_ref, a_ref, root_ref, bias_ref, o_ref):
  z = _DOT(h_ref[...], root_ref[...]) + bias_ref[...]
  z = z + a_ref[0] + a_ref[1]
  nrm = jnp.sqrt(jnp.sum(z * z, axis=1, keepdims=True))
  o_ref[...] = z / jnp.maximum(nrm, EPS_NORM)


def _tc3(h, a1, root1, bias1):
  return pl.pallas_call(
      _tc3_body,
      grid=(_GRID,),
      in_specs=[
          pl.BlockSpec((_BLK, F), lambda i: (i, 0)),
          pl.BlockSpec((NC, _BLK, F), lambda i: (0, i, 0)),
          pl.BlockSpec((F, F), lambda i: (0, 0)),
          pl.BlockSpec((F,), lambda i: (0,)),
      ],
      out_specs=pl.BlockSpec((_BLK, F), lambda i: (i, 0)),
      out_shape=jax.ShapeDtypeStruct((N, F), jnp.float32),
  )(h, a1, root1, bias1)


def kernel(x, edge_index, edge_type, comp0, bases0, root0, bias0,
           bn_gamma, bn_beta, bn_mean, bn_var, comp1, bases1, root1, bias1):
  src = edge_index[0]
  dst = edge_index[1]
  et = edge_type

  zero_hist = jnp.zeros((KPAD,), jnp.float32)
  zero_acc = jnp.zeros((N, F), jnp.float32)

  cnt2 = _cnt_call(dst, et, zero_hist)
  p0, inv = _tc1(x, comp0, bases0,
                 cnt2.reshape(NC, KPAD // 128, 128))
  invf = inv.reshape(KPAD)
  a0 = _pass_call(p0.reshape(R * N, F), invf, src, et, dst, zero_acc)

  bnscale = bn_gamma / jnp.sqrt(bn_var + EPS_BN)
  bnshift = bn_beta - bn_mean * bnscale
  p1, h = _tc2(x, a0, root0, bias0, bnscale, bnshift, comp1, bases1)
  a1 = _pass_call(p1.reshape(R * N, F), invf, src, et, dst, zero_acc)
  return _tc3(h, a1, root1, bias1)


# speedup vs baseline: 7.0761x; 1.1918x over previous
import functools

import jax
import jax.numpy as jnp
from jax import lax
from jax.experimental import pallas as pl
from jax.experimental.pallas import tpu as pltpu
from jax.experimental.pallas import tpu_sc as plsc

N = 10000
E = 320000
F = 128
R = 5
NB = 4
EPS_BN = 1e-5
EPS_NORM = 1e-12

NC = 2
NS = 16
NW = NC * NS
EPW = E // NW
CH = 80
NCHUNK = EPW // CH
SUP = 2000

KROWS = 3200
KPAD = KROWS * 16
ROW_A = 624
ROW_TAIL = N - NS * ROW_A
KR_PER_TILE = KROWS // NS

_mesh = plsc.VectorSubcoreMesh(core_axis_name="c", subcore_axis_name="s",
                               num_cores=NC, num_subcores=NS)
_SC_PARAMS = pltpu.CompilerParams(needs_layout_passes=False)


def _cnt_body(dst_hbm, et_hbm, zero_hbm, out_hbm,
              dst_v, et_v, key_v, ones_v, cnt_sh):
  c = lax.axis_index("c")
  s = lax.axis_index("s")
  wid = s * NC + c

  kst = pl.multiple_of(s * (KPAD // NS), 8)
  pltpu.sync_copy(zero_hbm.at[pl.ds(kst, KPAD // NS)],
                  cnt_sh.at[pl.ds(kst, KPAD // NS)])
  for g in range(CH // 16):
    ones_v[pl.ds(g * 16, 16)] = jnp.full((16,), 1.0, jnp.float32)
  plsc.subcore_barrier()

  def chunk(t, carry):
    base = pl.multiple_of(wid * EPW + t * CH, 8)
    pltpu.sync_copy(dst_hbm.at[pl.ds(base, CH)], dst_v)
    pltpu.sync_copy(et_hbm.at[pl.ds(base, CH)], et_v)
    for g in range(CH // 16):
      d16 = dst_v[pl.ds(g * 16, 16)]
      e16 = et_v[pl.ds(g * 16, 16)]
      key_v[pl.ds(g * 16, 16)] = d16 * R + e16
    pltpu.sync_copy(ones_v, cnt_sh.at[key_v], add=True)
    return carry

  lax.fori_loop(0, NCHUNK, chunk, 0)
  plsc.subcore_barrier()

  pltpu.sync_copy(cnt_sh.at[pl.ds(kst, KPAD // NS)],
                  out_hbm.at[c, pl.ds(kst, KPAD // NS)])


_cnt_call = pl.kernel(
    _cnt_body,
    out_type=jax.ShapeDtypeStruct((NC, KPAD), jnp.float32),
    mesh=_mesh,
    compiler_params=_SC_PARAMS,
    scratch_types=[
        pltpu.VMEM((CH,), jnp.int32),
        pltpu.VMEM((CH,), jnp.int32),
        pltpu.VMEM((CH,), jnp.int32),
        pltpu.VMEM((CH,), jnp.float32),
        pltpu.VMEM_SHARED((KPAD,), jnp.float32),
    ],
)


def _pass_body(p_hbm, inv_hbm, src_hbm, et_hbm, dst_hbm, zero_hbm, out_hbm,
               srcs_v, ets_v, dsts_v, ridx_v, key_v, dst_v, w_v, rows_v,
               inv_sh, acc_sh, sem, sem2):
  c = lax.axis_index("c")
  s = lax.axis_index("s")
  wid = s * NC + c

  rst = pl.multiple_of(s * ROW_A, 8)
  pltpu.sync_copy(zero_hbm.at[pl.ds(rst, ROW_A)],
                  acc_sh.at[pl.ds(rst, ROW_A)])
  @pl.when(s == NS - 1)
  def _zero_tail():
    pltpu.sync_copy(zero_hbm.at[pl.ds(NS * ROW_A, ROW_TAIL)],
                    acc_sh.at[pl.ds(NS * ROW_A, ROW_TAIL)])
  @pl.when(s == 0)
  def _stage_inv():
    pltpu.sync_copy(inv_hbm, inv_sh)
  plsc.subcore_barrier()

  iota16 = lax.iota(jnp.int32, 16)

  def sup(S, carry):
    base = pl.multiple_of(wid * EPW + S * SUP, 8)
    pltpu.sync_copy(src_hbm.at[pl.ds(base, SUP)], srcs_v)
    pltpu.sync_copy(et_hbm.at[pl.ds(base, SUP)], ets_v)
    pltpu.sync_copy(dst_hbm.at[pl.ds(base, SUP)], dsts_v)
    for g in range(SUP // 16):
      s16 = srcs_v[pl.ds(g * 16, 16)]
      e16 = ets_v[pl.ds(g * 16, 16)]
      d16 = dsts_v[pl.ds(g * 16, 16)]
      ridx_v[pl.ds(g * 16, 16)] = e16 * N + s16
      key_v[pl.ds(g * 16, 16)] = d16 * R + e16

    def chunk(t, carry2):
      off = pl.multiple_of(t * CH, 8)
      rows_dma = pltpu.async_copy(p_hbm.at[ridx_v.at[pl.ds(off, CH)]],
                                  rows_v, sem)
      w_dma = pltpu.async_copy(inv_sh.at[key_v.at[pl.ds(off, CH)]], w_v, sem2)
      for g2 in range(CH // 16):
        dst_v[pl.ds(g2 * 16, 16)] = plsc.load_gather(
            dsts_v, [iota16 + (t * CH + g2 * 16)])
      w_dma.wait()
      rows_dma.wait()
      def scale(i, carry3):
        for u in range(2):
          rsplat = jnp.full((16,), i * 2 + u, jnp.int32)
          w16 = plsc.load_gather(w_v, [rsplat])
          for j in range(F // 16):
            col = iota16 + (j * 16)
            v = plsc.load_gather(rows_v, [rsplat, col])
            plsc.store_scatter(rows_v, [rsplat, col], v * w16)
        return carry3
      lax.fori_loop(0, CH // 2, scale, 0)
      pltpu.sync_copy(rows_v, acc_sh.at[dst_v], add=True)
      return carry2

    lax.fori_loop(0, SUP // CH, chunk, 0)
    return carry

  lax.fori_loop(0, EPW // SUP, sup, 0)
  plsc.subcore_barrier()

  pltpu.sync_copy(acc_sh.at[pl.ds(rst, ROW_A)],
                  out_hbm.at[c, pl.ds(rst, ROW_A)])
  @pl.when(s == NS - 1)
  def _out_tail():
    pltpu.sync_copy(acc_sh.at[pl.ds(NS * ROW_A, ROW_TAIL)],
                    out_hbm.at[c, pl.ds(NS * ROW_A, ROW_TAIL)])


_pass_call = pl.kernel(
    _pass_body,
    out_type=jax.ShapeDtypeStruct((NC, N, F), jnp.float32),
    mesh=_mesh,
    compiler_params=_SC_PARAMS,
    scratch_types=[
        pltpu.VMEM((SUP,), jnp.int32),
        pltpu.VMEM((SUP,), jnp.int32),
        pltpu.VMEM((SUP,), jnp.int32),
        pltpu.VMEM((SUP,), jnp.int32),
        pltpu.VMEM((SUP,), jnp.int32),
        pltpu.VMEM((CH,), jnp.int32),
        pltpu.VMEM((CH,), jnp.float32),
        pltpu.VMEM((CH, F), jnp.float32),
        pltpu.VMEM_SHARED((KPAD,), jnp.float32),
        pltpu.VMEM_SHARED((N, F), jnp.float32),
        pltpu.SemaphoreType.DMA,
        pltpu.SemaphoreType.DMA,
    ],
)


_BLK = 1000
_GRID = N // _BLK
_DOT = functools.partial(lax.dot, precision=lax.Precision.HIGHEST,
                         preferred_element_type=jnp.float32)


def _tc1_body(x_ref, comp_ref, bases_ref, cnt_ref, p_ref, inv_ref):
  csum = cnt_ref[0] + cnt_ref[1]
  inv_ref[...] = 1.0 / jnp.maximum(csum, 1.0)
  xb = x_ref[...]
  for r in range(R):
    w = (comp_ref[r, 0] * bases_ref[0] + comp_ref[r, 1] * bases_ref[1]
         + comp_ref[r, 2] * bases_ref[2] + comp_ref[r, 3] * bases_ref[3])
    p_ref[r] = _DOT(xb, w)


def _tc1(x, comp0, bases0, cnt2):
  return pl.pallas_call(
      _tc1_body,
      grid=(_GRID,),
      in_specs=[
          pl.BlockSpec((_BLK, F), lambda i: (i, 0)),
          pl.BlockSpec((R, NB), lambda i: (0, 0)),
          pl.BlockSpec((NB, F, F), lambda i: (0, 0, 0)),
          pl.BlockSpec((NC, KPAD // 128, 128), lambda i: (0, 0, 0)),
      ],
      out_specs=[
          pl.BlockSpec((R, _BLK, F), lambda i: (0, i, 0)),
          pl.BlockSpec((KPAD // 128, 128), lambda i: (0, 0)),
      ],
      out_shape=[
          jax.ShapeDtypeStruct((R, N, F), jnp.float32),
          jax.ShapeDtypeStruct((KPAD // 128, 128), jnp.float32),
      ],
  )(x, comp0, bases0, cnt2)


def _tc2_body(x_ref, a_ref, root_ref, bias_ref, sc_ref, sh_ref,
              comp_ref, bases_ref, p_ref, h_ref):
  z = _DOT(x_ref[...], root_ref[...]) + bias_ref[...]
  z = z + a_ref[0] + a_ref[1]
  h = jnp.maximum(z * sc_ref[...] + sh_ref[...], 0.0)
  h_ref[...] = h
  for r in range(R):
    w = (comp_ref[r, 0] * bases_ref[0] + comp_ref[r, 1] * bases_ref[1]
         + comp_ref[r, 2] * bases_ref[2] + comp_ref[r, 3] * bases_ref[3])
    p_ref[r] = _DOT(h, w)


def _tc2(x, a0, root0, bias0, bnscale, bnshift, comp1, bases1):
  return pl.pallas_call(
      _tc2_body,
      grid=(_GRID,),
      in_specs=[
          pl.BlockSpec((_BLK, F), lambda i: (i, 0)),
          pl.BlockSpec((NC, _BLK, F), lambda i: (0, i, 0)),
          pl.BlockSpec((F, F), lambda i: (0, 0)),
          pl.BlockSpec((F,), lambda i: (0,)),
          pl.BlockSpec((F,), lambda i: (0,)),
          pl.BlockSpec((F,), lambda i: (0,)),
          pl.BlockSpec((R, NB), lambda i: (0, 0)),
          pl.BlockSpec((NB, F, F), lambda i: (0, 0, 0)),
      ],
      out_specs=[
          pl.BlockSpec((R, _BLK, F), lambda i: (0, i, 0)),
          pl.BlockSpec((_BLK, F), lambda i: (i, 0)),
      ],
      out_shape=[
          jax.ShapeDtypeStruct((R, N, F), jnp.float32),
          jax.ShapeDtypeStruct((N, F), jnp.float32),
      ],
  )(x, a0, root0, bias0, bnscale, bnshift, comp1, bases1)


def _tc3_body(h_ref, a_ref, root_ref, bias_ref, o_ref):
  z = _DOT(h_ref[...], root_ref[...]) + bias_ref[...]
  z = z + a_ref[0] + a_ref[1]
  nrm = jnp.sqrt(jnp.sum(z * z, axis=1, keepdims=True))
  o_ref[...] = z / jnp.maximum(nrm, EPS_NORM)


def _tc3(h, a1, root1, bias1):
  return pl.pallas_call(
      _tc3_body,
      grid=(_GRID,),
      in_specs=[
          pl.BlockSpec((_BLK, F), lambda i: (i, 0)),
          pl.BlockSpec((NC, _BLK, F), lambda i: (0, i, 0)),
          pl.BlockSpec((F, F), lambda i: (0, 0)),
          pl.BlockSpec((F,), lambda i: (0,)),
      ],
      out_specs=pl.BlockSpec((_BLK, F), lambda i: (i, 0)),
      out_shape=jax.ShapeDtypeStruct((N, F), jnp.float32),
  )(h, a1, root1, bias1)


def kernel(x, edge_index, edge_type, comp0, bases0, root0, bias0,
           bn_gamma, bn_beta, bn_mean, bn_var, comp1, bases1, root1, bias1):
  src = edge_index[0]
  dst = edge_index[1]
  et = edge_type

  zero_hist = jnp.zeros((KPAD,), jnp.float32)
  zero_acc = jnp.zeros((N, F), jnp.float32)

  cnt2 = _cnt_call(dst, et, zero_hist)
  p0, inv = _tc1(x, comp0, bases0,
                 cnt2.reshape(NC, KPAD // 128, 128))
  invf = inv.reshape(KPAD)
  a0 = _pass_call(p0.reshape(R * N, F), invf, src, et, dst, zero_acc)

  bnscale = bn_gamma / jnp.sqrt(bn_var + EPS_BN)
  bnshift = bn_beta - bn_mean * bnscale
  p1, h = _tc2(x, a0, root0, bias0, bnscale, bnshift, comp1, bases1)
  a1 = _pass_call(p1.reshape(R * N, F), invf, src, et, dst, zero_acc)
  return _tc3(h, a1, root1, bias1)


# speedup vs baseline: 8.3125x; 1.1747x over previous
import functools

import jax
import jax.numpy as jnp
from jax import lax
from jax.experimental import pallas as pl
from jax.experimental.pallas import tpu as pltpu
from jax.experimental.pallas import tpu_sc as plsc

N = 10000
E = 320000
F = 128
R = 5
NB = 4
EPS_BN = 1e-5
EPS_NORM = 1e-12

NC = 2
NS = 16
NW = NC * NS
EPW = E // NW
CH = 80
NCHUNK = EPW // CH
SUP = 2000

KROWS = 3200
KPAD = KROWS * 16
ROW_A = 624
ROW_TAIL = N - NS * ROW_A
KR_PER_TILE = KROWS // NS

_mesh = plsc.VectorSubcoreMesh(core_axis_name="c", subcore_axis_name="s",
                               num_cores=NC, num_subcores=NS)
_SC_PARAMS = pltpu.CompilerParams(needs_layout_passes=False)


def _cnt_body(dst_hbm, et_hbm, zero_hbm, out_hbm,
              dst_v, et_v, key_v, ones_v, cnt_sh):
  c = lax.axis_index("c")
  s = lax.axis_index("s")
  wid = s * NC + c

  kst = pl.multiple_of(s * (KPAD // NS), 8)
  pltpu.sync_copy(zero_hbm.at[pl.ds(kst, KPAD // NS)],
                  cnt_sh.at[pl.ds(kst, KPAD // NS)])
  for g in range(CH // 16):
    ones_v[pl.ds(g * 16, 16)] = jnp.full((16,), 1.0, jnp.float32)
  plsc.subcore_barrier()

  def chunk(t, carry):
    base = pl.multiple_of(wid * EPW + t * CH, 8)
    pltpu.sync_copy(dst_hbm.at[pl.ds(base, CH)], dst_v)
    pltpu.sync_copy(et_hbm.at[pl.ds(base, CH)], et_v)
    for g in range(CH // 16):
      d16 = dst_v[pl.ds(g * 16, 16)]
      e16 = et_v[pl.ds(g * 16, 16)]
      key_v[pl.ds(g * 16, 16)] = d16 * R + e16
    pltpu.sync_copy(ones_v, cnt_sh.at[key_v], add=True)
    return carry

  lax.fori_loop(0, NCHUNK, chunk, 0)
  plsc.subcore_barrier()

  pltpu.sync_copy(cnt_sh.at[pl.ds(kst, KPAD // NS)],
                  out_hbm.at[c, pl.ds(kst, KPAD // NS)])


_cnt_call = pl.kernel(
    _cnt_body,
    out_type=jax.ShapeDtypeStruct((NC, KPAD), jnp.float32),
    mesh=_mesh,
    compiler_params=_SC_PARAMS,
    scratch_types=[
        pltpu.VMEM((CH,), jnp.int32),
        pltpu.VMEM((CH,), jnp.int32),
        pltpu.VMEM((CH,), jnp.int32),
        pltpu.VMEM((CH,), jnp.float32),
        pltpu.VMEM_SHARED((KPAD,), jnp.float32),
    ],
)


def _pass_body(p_hbm, inv_hbm, src_hbm, et_hbm, dst_hbm, zero_hbm, out_hbm,
               srcs_v, ets_v, dsts_v, ridx_v, key_v, dst_v,
               w_a, w_b, rows_a, rows_b,
               inv_sh, acc_sh, sem_r0, sem_r1, sem_w0, sem_w1):
  c = lax.axis_index("c")
  s = lax.axis_index("s")
  wid = s * NC + c

  rst = pl.multiple_of(s * ROW_A, 8)
  pltpu.sync_copy(zero_hbm.at[pl.ds(rst, ROW_A)],
                  acc_sh.at[pl.ds(rst, ROW_A)])
  @pl.when(s == NS - 1)
  def _zero_tail():
    pltpu.sync_copy(zero_hbm.at[pl.ds(NS * ROW_A, ROW_TAIL)],
                    acc_sh.at[pl.ds(NS * ROW_A, ROW_TAIL)])
  @pl.when(s == 0)
  def _stage_inv():
    pltpu.sync_copy(inv_hbm, inv_sh)
  plsc.subcore_barrier()

  iota16 = lax.iota(jnp.int32, 16)

  def sup(S, carry):
    base = pl.multiple_of(wid * EPW + S * SUP, 8)
    pltpu.sync_copy(src_hbm.at[pl.ds(base, SUP)], srcs_v)
    pltpu.sync_copy(et_hbm.at[pl.ds(base, SUP)], ets_v)
    pltpu.sync_copy(dst_hbm.at[pl.ds(base, SUP)], dsts_v)
    for g in range(SUP // 16):
      s16 = srcs_v[pl.ds(g * 16, 16)]
      e16 = ets_v[pl.ds(g * 16, 16)]
      d16 = dsts_v[pl.ds(g * 16, 16)]
      ridx_v[pl.ds(g * 16, 16)] = e16 * N + s16
      key_v[pl.ds(g * 16, 16)] = d16 * R + e16

    rbufs = (rows_a, rows_b)
    wbufs = (w_a, w_b)
    rsems = (sem_r0, sem_r1)
    wsems = (sem_w0, sem_w1)

    def start(t):
      p = t % 2
      off = t * CH
      rd = pltpu.async_copy(p_hbm.at[ridx_v.at[pl.ds(off, CH)]],
                            rbufs[p], rsems[p])
      wd = pltpu.async_copy(inv_sh.at[key_v.at[pl.ds(off, CH)]],
                            wbufs[p], wsems[p])
      return rd, wd

    def scale_buf(rv, wv):
      def scale(i, carry3):
        for u in range(2):
          rsplat = jnp.full((16,), i * 2 + u, jnp.int32)
          w16 = plsc.load_gather(wv, [rsplat])
          for j in range(F // 16):
            col = iota16 + (j * 16)
            v = plsc.load_gather(rv, [rsplat, col])
            plsc.store_scatter(rv, [rsplat, col], v * w16)
        return carry3
      lax.fori_loop(0, CH // 2, scale, 0)

    pend = start(0)
    for t in range(SUP // CH):
      nxt = start(t + 1) if t + 1 < SUP // CH else None
      for g2 in range(CH // 16):
        dst_v[pl.ds(g2 * 16, 16)] = plsc.load_gather(
            dsts_v, [iota16 + (t * CH + g2 * 16)])
      pend[1].wait()
      pend[0].wait()
      rv = rbufs[t % 2]
      scale_buf(rv, wbufs[t % 2])
      pltpu.sync_copy(rv, acc_sh.at[dst_v], add=True)
      pend = nxt
    return carry

  lax.fori_loop(0, EPW // SUP, sup, 0)
  plsc.subcore_barrier()

  pltpu.sync_copy(acc_sh.at[pl.ds(rst, ROW_A)],
                  out_hbm.at[c, pl.ds(rst, ROW_A)])
  @pl.when(s == NS - 1)
  def _out_tail():
    pltpu.sync_copy(acc_sh.at[pl.ds(NS * ROW_A, ROW_TAIL)],
                    out_hbm.at[c, pl.ds(NS * ROW_A, ROW_TAIL)])


_pass_call = pl.kernel(
    _pass_body,
    out_type=jax.ShapeDtypeStruct((NC, N, F), jnp.float32),
    mesh=_mesh,
    compiler_params=_SC_PARAMS,
    scratch_types=[
        pltpu.VMEM((SUP,), jnp.int32),
        pltpu.VMEM((SUP,), jnp.int32),
        pltpu.VMEM((SUP,), jnp.int32),
        pltpu.VMEM((SUP,), jnp.int32),
        pltpu.VMEM((SUP,), jnp.int32),
        pltpu.VMEM((CH,), jnp.int32),
        pltpu.VMEM((CH,), jnp.float32),
        pltpu.VMEM((CH,), jnp.float32),
        pltpu.VMEM((CH, F), jnp.float32),
        pltpu.VMEM((CH, F), jnp.float32),
        pltpu.VMEM_SHARED((KPAD,), jnp.float32),
        pltpu.VMEM_SHARED((N, F), jnp.float32),
        pltpu.SemaphoreType.DMA,
        pltpu.SemaphoreType.DMA,
        pltpu.SemaphoreType.DMA,
        pltpu.SemaphoreType.DMA,
    ],
)


_BLK = 1000
_GRID = N // _BLK
_DOT = functools.partial(lax.dot, precision=lax.Precision.HIGHEST,
                         preferred_element_type=jnp.float32)


def _tc1_body(x_ref, comp_ref, bases_ref, cnt_ref, p_ref, inv_ref):
  csum = cnt_ref[0] + cnt_ref[1]
  inv_ref[...] = 1.0 / jnp.maximum(csum, 1.0)
  xb = x_ref[...]
  for r in range(R):
    w = (comp_ref[r, 0] * bases_ref[0] + comp_ref[r, 1] * bases_ref[1]
         + comp_ref[r, 2] * bases_ref[2] + comp_ref[r, 3] * bases_ref[3])
    p_ref[r] = _DOT(xb, w)


def _tc1(x, comp0, bases0, cnt2):
  return pl.pallas_call(
      _tc1_body,
      grid=(_GRID,),
      in_specs=[
          pl.BlockSpec((_BLK, F), lambda i: (i, 0)),
          pl.BlockSpec((R, NB), lambda i: (0, 0)),
          pl.BlockSpec((NB, F, F), lambda i: (0, 0, 0)),
          pl.BlockSpec((NC, KPAD // 128, 128), lambda i: (0, 0, 0)),
      ],
      out_specs=[
          pl.BlockSpec((R, _BLK, F), lambda i: (0, i, 0)),
          pl.BlockSpec((KPAD // 128, 128), lambda i: (0, 0)),
      ],
      out_shape=[
          jax.ShapeDtypeStruct((R, N, F), jnp.float32),
          jax.ShapeDtypeStruct((KPAD // 128, 128), jnp.float32),
      ],
  )(x, comp0, bases0, cnt2)


def _tc2_body(x_ref, a_ref, root_ref, bias_ref, sc_ref, sh_ref,
              comp_ref, bases_ref, p_ref, h_ref):
  z = _DOT(x_ref[...], root_ref[...]) + bias_ref[...]
  z = z + a_ref[0] + a_ref[1]
  h = jnp.maximum(z * sc_ref[...] + sh_ref[...], 0.0)
  h_ref[...] = h
  for r in range(R):
    w = (comp_ref[r, 0] * bases_ref[0] + comp_ref[r, 1] * bases_ref[1]
         + comp_ref[r, 2] * bases_ref[2] + comp_ref[r, 3] * bases_ref[3])
    p_ref[r] = _DOT(h, w)


def _tc2(x, a0, root0, bias0, bnscale, bnshift, comp1, bases1):
  return pl.pallas_call(
      _tc2_body,
      grid=(_GRID,),
      in_specs=[
          pl.BlockSpec((_BLK, F), lambda i: (i, 0)),
          pl.BlockSpec((NC, _BLK, F), lambda i: (0, i, 0)),
          pl.BlockSpec((F, F), lambda i: (0, 0)),
          pl.BlockSpec((F,), lambda i: (0,)),
          pl.BlockSpec((F,), lambda i: (0,)),
          pl.BlockSpec((F,), lambda i: (0,)),
          pl.BlockSpec((R, NB), lambda i: (0, 0)),
          pl.BlockSpec((NB, F, F), lambda i: (0, 0, 0)),
      ],
      out_specs=[
          pl.BlockSpec((R, _BLK, F), lambda i: (0, i, 0)),
          pl.BlockSpec((_BLK, F), lambda i: (i, 0)),
      ],
      out_shape=[
          jax.ShapeDtypeStruct((R, N, F), jnp.float32),
          jax.ShapeDtypeStruct((N, F), jnp.float32),
      ],
  )(x, a0, root0, bias0, bnscale, bnshift, comp1, bases1)


def _tc3_body(h_ref, a_ref, root_ref, bias_ref, o_ref):
  z = _DOT(h_ref[...], root_ref[...]) + bias_ref[...]
  z = z + a_ref[0] + a_ref[1]
  nrm = jnp.sqrt(jnp.sum(z * z, axis=1, keepdims=True))
  o_ref[...] = z / jnp.maximum(nrm, EPS_NORM)


def _tc3(h, a1, root1, bias1):
  return pl.pallas_call(
      _tc3_body,
      grid=(_GRID,),
      in_specs=[
          pl.BlockSpec((_BLK, F), lambda i: (i, 0)),
          pl.BlockSpec((NC, _BLK, F), lambda i: (0, i, 0)),
          pl.BlockSpec((F, F), lambda i: (0, 0)),
          pl.BlockSpec((F,), lambda i: (0,)),
      ],
      out_specs=pl.BlockSpec((_BLK, F), lambda i: (i, 0)),
      out_shape=jax.ShapeDtypeStruct((N, F), jnp.float32),
  )(h, a1, root1, bias1)


def kernel(x, edge_index, edge_type, comp0, bases0, root0, bias0,
           bn_gamma, bn_beta, bn_mean, bn_var, comp1, bases1, root1, bias1):
  src = edge_index[0]
  dst = edge_index[1]
  et = edge_type

  zero_hist = jnp.zeros((KPAD,), jnp.float32)
  zero_acc = jnp.zeros((N, F), jnp.float32)

  cnt2 = _cnt_call(dst, et, zero_hist)
  p0, inv = _tc1(x, comp0, bases0,
                 cnt2.reshape(NC, KPAD // 128, 128))
  invf = inv.reshape(KPAD)
  a0 = _pass_call(p0.reshape(R * N, F), invf, src, et, dst, zero_acc)

  bnscale = bn_gamma / jnp.sqrt(bn_var + EPS_BN)
  bnshift = bn_beta - bn_mean * bnscale
  p1, h = _tc2(x, a0, root0, bias0, bnscale, bnshift, comp1, bases1)
  a1 = _pass_call(p1.reshape(R * N, F), invf, src, et, dst, zero_acc)
  return _tc3(h, a1, root1, bias1)


# speedup vs baseline: 9.5574x; 1.1498x over previous
import functools

import jax
import jax.numpy as jnp
from jax import lax
from jax.experimental import pallas as pl
from jax.experimental.pallas import tpu as pltpu
from jax.experimental.pallas import tpu_sc as plsc

N = 10000
E = 320000
F = 128
R = 5
NB = 4
EPS_BN = 1e-5
EPS_NORM = 1e-12

NC = 2
NS = 16
NW = NC * NS
EPW = E // NW
CH = 80
NCHUNK = EPW // CH
SUP = 2000

KROWS = 3200
KPAD = KROWS * 16
ROW_A = 624
ROW_TAIL = N - NS * ROW_A
KR_PER_TILE = KROWS // NS

_mesh = plsc.VectorSubcoreMesh(core_axis_name="c", subcore_axis_name="s",
                               num_cores=NC, num_subcores=NS)
_SC_PARAMS = pltpu.CompilerParams(needs_layout_passes=False)


def _cnt_body(dst_hbm, et_hbm, zero_hbm, out_hbm,
              dst_v, et_v, key_v, ones_v, cnt_sh):
  c = lax.axis_index("c")
  s = lax.axis_index("s")
  wid = s * NC + c

  kst = pl.multiple_of(s * (KPAD // NS), 8)
  pltpu.sync_copy(zero_hbm.at[pl.ds(kst, KPAD // NS)],
                  cnt_sh.at[pl.ds(kst, KPAD // NS)])
  for g in range(CH // 16):
    ones_v[pl.ds(g * 16, 16)] = jnp.full((16,), 1.0, jnp.float32)
  plsc.subcore_barrier()

  def chunk(t, carry):
    base = pl.multiple_of(wid * EPW + t * CH, 8)
    pltpu.sync_copy(dst_hbm.at[pl.ds(base, CH)], dst_v)
    pltpu.sync_copy(et_hbm.at[pl.ds(base, CH)], et_v)
    for g in range(CH // 16):
      d16 = dst_v[pl.ds(g * 16, 16)]
      e16 = et_v[pl.ds(g * 16, 16)]
      key_v[pl.ds(g * 16, 16)] = d16 * R + e16
    pltpu.sync_copy(ones_v, cnt_sh.at[key_v], add=True)
    return carry

  lax.fori_loop(0, NCHUNK, chunk, 0)
  plsc.subcore_barrier()

  pltpu.sync_copy(cnt_sh.at[pl.ds(kst, KPAD // NS)],
                  out_hbm.at[c, pl.ds(kst, KPAD // NS)])


_cnt_call = pl.kernel(
    _cnt_body,
    out_type=jax.ShapeDtypeStruct((NC, KPAD), jnp.float32),
    mesh=_mesh,
    compiler_params=_SC_PARAMS,
    scratch_types=[
        pltpu.VMEM((CH,), jnp.int32),
        pltpu.VMEM((CH,), jnp.int32),
        pltpu.VMEM((CH,), jnp.int32),
        pltpu.VMEM((CH,), jnp.float32),
        pltpu.VMEM_SHARED((KPAD,), jnp.float32),
    ],
)


def _pass_body(p_hbm, inv_hbm, src_hbm, et_hbm, dst_hbm, zero_hbm, out_hbm,
               srcs_v, ets_v, dsts_v, ridx_v, key_v, dst_a, dst_b, dst_c,
               w_a, w_b, w_c, rows_a, rows_b, rows_c,
               inv_sh, acc_sh, sem_r0, sem_r1, sem_r2, sem_w0, sem_w1, sem_w2,
               sem_s0, sem_s1, sem_s2):
  c = lax.axis_index("c")
  s = lax.axis_index("s")
  wid = s * NC + c

  rst = pl.multiple_of(s * ROW_A, 8)
  pltpu.sync_copy(zero_hbm.at[pl.ds(rst, ROW_A)],
                  acc_sh.at[pl.ds(rst, ROW_A)])
  @pl.when(s == NS - 1)
  def _zero_tail():
    pltpu.sync_copy(zero_hbm.at[pl.ds(NS * ROW_A, ROW_TAIL)],
                    acc_sh.at[pl.ds(NS * ROW_A, ROW_TAIL)])
  @pl.when(s == 0)
  def _stage_inv():
    pltpu.sync_copy(inv_hbm, inv_sh)
  plsc.subcore_barrier()

  iota16 = lax.iota(jnp.int32, 16)

  def sup(S, carry):
    base = pl.multiple_of(wid * EPW + S * SUP, 8)
    pltpu.sync_copy(src_hbm.at[pl.ds(base, SUP)], srcs_v)
    pltpu.sync_copy(et_hbm.at[pl.ds(base, SUP)], ets_v)
    pltpu.sync_copy(dst_hbm.at[pl.ds(base, SUP)], dsts_v)
    for g in range(SUP // 16):
      s16 = srcs_v[pl.ds(g * 16, 16)]
      e16 = ets_v[pl.ds(g * 16, 16)]
      d16 = dsts_v[pl.ds(g * 16, 16)]
      ridx_v[pl.ds(g * 16, 16)] = e16 * N + s16
      key_v[pl.ds(g * 16, 16)] = d16 * R + e16

    rbufs = (rows_a, rows_b, rows_c)
    wbufs = (w_a, w_b, w_c)
    dbufs = (dst_a, dst_b, dst_c)
    rsems = (sem_r0, sem_r1, sem_r2)
    wsems = (sem_w0, sem_w1, sem_w2)
    ssems = (sem_s0, sem_s1, sem_s2)

    def start(t):
      p = t % 3
      off = t * CH
      rd = pltpu.async_copy(p_hbm.at[ridx_v.at[pl.ds(off, CH)]],
                            rbufs[p], rsems[p])
      wd = pltpu.async_copy(inv_sh.at[key_v.at[pl.ds(off, CH)]],
                            wbufs[p], wsems[p])
      return rd, wd

    def scale_buf(rv, wv):
      def scale(i, carry3):
        for u in range(4):
          rsplat = jnp.full((16,), i * 4 + u, jnp.int32)
          w16 = plsc.load_gather(wv, [rsplat])
          for j in range(F // 16):
            col = iota16 + (j * 16)
            v = plsc.load_gather(rv, [rsplat, col])
            plsc.store_scatter(rv, [rsplat, col], v * w16)
        return carry3
      lax.fori_loop(0, CH // 4, scale, 0)

    nch = SUP // CH
    pend = start(0)
    pend_scat = [None, None, None]
    for t in range(nch):
      nxt = None
      if t + 1 < nch:
        q = (t + 1) % 3
        if pend_scat[q] is not None:
          pend_scat[q].wait()
          pend_scat[q] = None
        nxt = start(t + 1)
      p = t % 3
      for g2 in range(CH // 16):
        dbufs[p][pl.ds(g2 * 16, 16)] = plsc.load_gather(
            dsts_v, [iota16 + (t * CH + g2 * 16)])
      pend[1].wait()
      pend[0].wait()
      scale_buf(rbufs[p], wbufs[p])
      pend_scat[p] = pltpu.async_copy(rbufs[p], acc_sh.at[dbufs[p]],
                                      ssems[p], add=True)
      pend = nxt
    for p in range(3):
      if pend_scat[p] is not None:
        pend_scat[p].wait()
    return carry

  lax.fori_loop(0, EPW // SUP, sup, 0)
  plsc.subcore_barrier()

  pltpu.sync_copy(acc_sh.at[pl.ds(rst, ROW_A)],
                  out_hbm.at[c, pl.ds(rst, ROW_A)])
  @pl.when(s == NS - 1)
  def _out_tail():
    pltpu.sync_copy(acc_sh.at[pl.ds(NS * ROW_A, ROW_TAIL)],
                    out_hbm.at[c, pl.ds(NS * ROW_A, ROW_TAIL)])


_pass_call = pl.kernel(
    _pass_body,
    out_type=jax.ShapeDtypeStruct((NC, N, F), jnp.float32),
    mesh=_mesh,
    compiler_params=_SC_PARAMS,
    scratch_types=[
        pltpu.VMEM((SUP,), jnp.int32),
        pltpu.VMEM((SUP,), jnp.int32),
        pltpu.VMEM((SUP,), jnp.int32),
        pltpu.VMEM((SUP,), jnp.int32),
        pltpu.VMEM((SUP,), jnp.int32),
        pltpu.VMEM((CH,), jnp.int32),
        pltpu.VMEM((CH,), jnp.int32),
        pltpu.VMEM((CH,), jnp.int32),
        pltpu.VMEM((CH,), jnp.float32),
        pltpu.VMEM((CH,), jnp.float32),
        pltpu.VMEM((CH,), jnp.float32),
        pltpu.VMEM((CH, F), jnp.float32),
        pltpu.VMEM((CH, F), jnp.float32),
        pltpu.VMEM((CH, F), jnp.float32),
        pltpu.VMEM_SHARED((KPAD,), jnp.float32),
        pltpu.VMEM_SHARED((N, F), jnp.float32),
    ] + [pltpu.SemaphoreType.DMA] * 9,
)


_BLK = 1000
_GRID = N // _BLK
_DOT = functools.partial(lax.dot, precision=lax.Precision.HIGHEST,
                         preferred_element_type=jnp.float32)


def _tc1_body(x_ref, comp_ref, bases_ref, cnt_ref, p_ref, inv_ref):
  csum = cnt_ref[0] + cnt_ref[1]
  inv_ref[...] = 1.0 / jnp.maximum(csum, 1.0)
  xb = x_ref[...]
  for r in range(R):
    w = (comp_ref[r, 0] * bases_ref[0] + comp_ref[r, 1] * bases_ref[1]
         + comp_ref[r, 2] * bases_ref[2] + comp_ref[r, 3] * bases_ref[3])
    p_ref[r] = _DOT(xb, w)


def _tc1(x, comp0, bases0, cnt2):
  return pl.pallas_call(
      _tc1_body,
      grid=(_GRID,),
      in_specs=[
          pl.BlockSpec((_BLK, F), lambda i: (i, 0)),
          pl.BlockSpec((R, NB), lambda i: (0, 0)),
          pl.BlockSpec((NB, F, F), lambda i: (0, 0, 0)),
          pl.BlockSpec((NC, KPAD // 128, 128), lambda i: (0, 0, 0)),
      ],
      out_specs=[
          pl.BlockSpec((R, _BLK, F), lambda i: (0, i, 0)),
          pl.BlockSpec((KPAD // 128, 128), lambda i: (0, 0)),
      ],
      out_shape=[
          jax.ShapeDtypeStruct((R, N, F), jnp.float32),
          jax.ShapeDtypeStruct((KPAD // 128, 128), jnp.float32),
      ],
  )(x, comp0, bases0, cnt2)


def _tc2_body(x_ref, a_ref, root_ref, bias_ref, sc_ref, sh_ref,
              comp_ref, bases_ref, p_ref, h_ref):
  z = _DOT(x_ref[...], root_ref[...]) + bias_ref[...]
  z = z + a_ref[0] + a_ref[1]
  h = jnp.maximum(z * sc_ref[...] + sh_ref[...], 0.0)
  h_ref[...] = h
  for r in range(R):
    w = (comp_ref[r, 0] * bases_ref[0] + comp_ref[r, 1] * bases_ref[1]
         + comp_ref[r, 2] * bases_ref[2] + comp_ref[r, 3] * bases_ref[3])
    p_ref[r] = _DOT(h, w)


def _tc2(x, a0, root0, bias0, bnscale, bnshift, comp1, bases1):
  return pl.pallas_call(
      _tc2_body,
      grid=(_GRID,),
      in_specs=[
          pl.BlockSpec((_BLK, F), lambda i: (i, 0)),
          pl.BlockSpec((NC, _BLK, F), lambda i: (0, i, 0)),
          pl.BlockSpec((F, F), lambda i: (0, 0)),
          pl.BlockSpec((F,), lambda i: (0,)),
          pl.BlockSpec((F,), lambda i: (0,)),
          pl.BlockSpec((F,), lambda i: (0,)),
          pl.BlockSpec((R, NB), lambda i: (0, 0)),
          pl.BlockSpec((NB, F, F), lambda i: (0, 0, 0)),
      ],
      out_specs=[
          pl.BlockSpec((R, _BLK, F), lambda i: (0, i, 0)),
          pl.BlockSpec((_BLK, F), lambda i: (i, 0)),
      ],
      out_shape=[
          jax.ShapeDtypeStruct((R, N, F), jnp.float32),
          jax.ShapeDtypeStruct((N, F), jnp.float32),
      ],
  )(x, a0, root0, bias0, bnscale, bnshift, comp1, bases1)


def _tc3_body(h_ref, a_ref, root_ref, bias_ref, o_ref):
  z = _DOT(h_ref[...], root_ref[...]) + bias_ref[...]
  z = z + a_ref[0] + a_ref[1]
  nrm = jnp.sqrt(jnp.sum(z * z, axis=1, keepdims=True))
  o_ref[...] = z / jnp.maximum(nrm, EPS_NORM)


def _tc3(h, a1, root1, bias1):
  return pl.pallas_call(
      _tc3_body,
      grid=(_GRID,),
      in_specs=[
          pl.BlockSpec((_BLK, F), lambda i: (i, 0)),
          pl.BlockSpec((NC, _BLK, F), lambda i: (0, i, 0)),
          pl.BlockSpec((F, F), lambda i: (0, 0)),
          pl.BlockSpec((F,), lambda i: (0,)),
      ],
      out_specs=pl.BlockSpec((_BLK, F), lambda i: (i, 0)),
      out_shape=jax.ShapeDtypeStruct((N, F), jnp.float32),
  )(h, a1, root1, bias1)


def kernel(x, edge_index, edge_type, comp0, bases0, root0, bias0,
           bn_gamma, bn_beta, bn_mean, bn_var, comp1, bases1, root1, bias1):
  src = edge_index[0]
  dst = edge_index[1]
  et = edge_type

  zero_hist = jnp.zeros((KPAD,), jnp.float32)
  zero_acc = jnp.zeros((N, F), jnp.float32)

  cnt2 = _cnt_call(dst, et, zero_hist)
  p0, inv = _tc1(x, comp0, bases0,
                 cnt2.reshape(NC, KPAD // 128, 128))
  invf = inv.reshape(KPAD)
  a0 = _pass_call(p0.reshape(R * N, F), invf, src, et, dst, zero_acc)

  bnscale = bn_gamma / jnp.sqrt(bn_var + EPS_BN)
  bnshift = bn_beta - bn_mean * bnscale
  p1, h = _tc2(x, a0, root0, bias0, bnscale, bnshift, comp1, bases1)
  a1 = _pass_call(p1.reshape(R * N, F), invf, src, et, dst, zero_acc)
  return _tc3(h, a1, root1, bias1)
